# Initial kernel scaffold; baseline (speedup 1.0000x reference)
import jax
import jax.numpy as jnp
from jax.experimental import pallas as pl


def kernel(node_embeddings, Wz, bz, Wr, br, Wh, bh, Lz_w, Lz_b, Lr_w, Lr_b, Lh_w, Lh_b, P1_w, P1_b, P2_w, P2_b, edge_index, node_ids):
    raise NotImplementedError("write your pallas kernel here")


# speedup vs baseline: 35.4493x; 35.4493x over previous
import functools

import jax
import jax.numpy as jnp
from jax import lax
from jax.experimental import pallas as pl
from jax.experimental.pallas import tpu as pltpu
from jax.experimental.pallas import tpu_sc as plsc

N = 10000
D = 128
HD = D // 2
E = 320000
NB = 4096
NP = 10240
RPT = NP // 16
CA = 79
CC = 158
EPAD = 32 * CA * 128
RB = 1280
GRID = NP // RB

_f32 = jnp.float32
_SC_PARAMS = pltpu.CompilerParams(use_tc_tiling_on_sc=False)


def _sc_mesh():
    return plsc.VectorSubcoreMesh(
        core_axis_name="c", subcore_axis_name="s", num_cores=2, num_subcores=16
    )


def _deg_kernel():
    @functools.partial(
        pl.kernel,
        out_type=jax.ShapeDtypeStruct((2 * NP, 16), _f32),
        mesh=_sc_mesh(),
        scratch_types=[
            pltpu.VMEM((CA, 128), jnp.int32),
            pltpu.VMEM((128, 16), _f32),
            pltpu.VMEM_SHARED((NP, 16), _f32),
        ],
        compiler_params=_SC_PARAMS,
    )
    def deg_k(dst_hbm, ones_hbm, zeros_hbm, out_hbm, dst_v, ones_v, acc):
        c = lax.axis_index("c")
        s = lax.axis_index("s")
        wid = c * 16 + s
        pltpu.sync_copy(dst_hbm.at[wid], dst_v)
        pltpu.sync_copy(ones_hbm, ones_v)
        pltpu.sync_copy(zeros_hbm.at[pl.ds(s * RPT, RPT)],
                        acc.at[pl.ds(s * RPT, RPT)])
        plsc.subcore_barrier()

        def body(j, carry):
            pltpu.sync_copy(ones_v, acc.at[dst_v.at[j]], add=True)
            return carry

        lax.fori_loop(0, CA, body, 0)
        plsc.subcore_barrier()
        pltpu.sync_copy(acc.at[pl.ds(s * RPT, RPT)],
                        out_hbm.at[pl.ds(c * NP + s * RPT, RPT)])

    return deg_k


def _b_body(x_ref, wz_ref, wh_ref, bz_ref, bh_ref, degp_ref, y_ref, base_ref):
    deg = degp_ref[0, :, 0] + degp_ref[1, :, 0] + 1.0
    dinv = lax.rsqrt(deg)[:, None]
    x = x_ref[...]
    for k, (w_ref, b_ref) in enumerate(((wz_ref, bz_ref), (wh_ref, bh_ref))):
        xw = jnp.dot(x, w_ref[...], preferred_element_type=_f32)
        y_ref[k] = xw * dinv
        base_ref[k] = xw * (dinv * dinv) + b_ref[...]


def _conv_kernel():
    @functools.partial(
        pl.kernel,
        out_type=jax.ShapeDtypeStruct((2 * NP, D), _f32),
        mesh=_sc_mesh(),
        scratch_types=[
            pltpu.VMEM((CC, 128), jnp.int32),
            pltpu.VMEM((CC, 128), jnp.int32),
            pltpu.VMEM((128, HD), _f32),
            pltpu.VMEM((128, HD), _f32),
            pltpu.VMEM_SHARED((NP, HD), _f32),
            pltpu.SemaphoreType.DMA,
            pltpu.SemaphoreType.DMA,
        ],
        compiler_params=_SC_PARAMS,
    )
    def conv_k(y_hbm, src_hbm, dst_hbm, zeros_hbm, out_hbm,
               src_v, dst_v, g0, g1, acc, s0, s1):
        c = lax.axis_index("c")
        s = lax.axis_index("s")
        pltpu.sync_copy(dst_hbm.at[s], dst_v)
        for p in range(2):
            q = c * 2 + p
            pltpu.sync_copy(src_hbm.at[q * 16 + s], src_v)
            pltpu.sync_copy(zeros_hbm.at[pl.ds(s * RPT, RPT)],
                            acc.at[pl.ds(s * RPT, RPT)])
            plsc.subcore_barrier()

            pltpu.async_copy(y_hbm.at[src_v.at[0]], g0, s0)

            def pair(i, carry):
                j = 2 * i
                pltpu.async_copy(y_hbm.at[src_v.at[j + 1]], g1, s1)
                pltpu.make_async_copy(y_hbm.at[src_v.at[j]], g0, s0).wait()
                pltpu.sync_copy(g0, acc.at[dst_v.at[j]], add=True)
                nxt = jnp.minimum(j + 2, CC - 1)
                pltpu.async_copy(y_hbm.at[src_v.at[nxt]], g0, s0)
                pltpu.make_async_copy(y_hbm.at[src_v.at[j + 1]], g1, s1).wait()
                pltpu.sync_copy(g1, acc.at[dst_v.at[j + 1]], add=True)
                return carry

            lax.fori_loop(0, CC // 2, pair, 0)
            pltpu.make_async_copy(y_hbm.at[src_v.at[CC - 1]], g0, s0).wait()
            plsc.subcore_barrier()
            pltpu.sync_copy(
                acc.at[pl.ds(s * RPT, RPT)],
                out_hbm.at[pl.ds(c * NP + s * RPT, RPT), pl.ds(p * HD, HD)])
            plsc.subcore_barrier()

    return conv_k


def _d_body(acc_ref, base_ref, degp_ref, lz_ref, lzb_ref, lh_ref, lhb_ref,
            p1_ref, p1b_ref, p2_ref, p2b_ref, out_ref):
    deg = degp_ref[0, :, 0] + degp_ref[1, :, 0] + 1.0
    dinv = lax.rsqrt(deg)[:, None]
    cz = acc_ref[0] * dinv + base_ref[0]
    z = jax.nn.sigmoid(
        jnp.dot(cz, lz_ref[...], preferred_element_type=_f32) + lzb_ref[...])
    ch = acc_ref[1] * dinv + base_ref[1]
    ht = jnp.tanh(
        jnp.dot(ch, lh_ref[...], preferred_element_type=_f32) + lhb_ref[...])
    hn = (1.0 - z) * ht
    h = jax.nn.relu(
        jnp.dot(hn, p1_ref[...], preferred_element_type=_f32) + p1b_ref[...])
    out_ref[...] = (
        jnp.dot(h, p2_ref[...], preferred_element_type=_f32) + p2b_ref[...])


def _gather_kernel():
    @functools.partial(
        pl.kernel,
        out_type=jax.ShapeDtypeStruct((NB, 16), _f32),
        mesh=_sc_mesh(),
        scratch_types=[
            pltpu.VMEM((128,), jnp.int32),
            pltpu.VMEM((128, 16), _f32),
            pltpu.SemaphoreType.DMA,
        ],
        compiler_params=_SC_PARAMS,
    )
    def gather_k(tab_hbm, ids_hbm, out_hbm, idx_v, rows_v, sem):
        wid = lax.axis_index("c") * 16 + lax.axis_index("s")
        pltpu.sync_copy(ids_hbm.at[wid], idx_v)
        pltpu.async_copy(tab_hbm.at[idx_v], rows_v, sem).wait()
        pltpu.sync_copy(rows_v, out_hbm.at[pl.ds(wid * 128, 128)])

    return gather_k


def kernel(node_embeddings, Wz, bz, Wr, br, Wh, bh, Lz_w, Lz_b, Lr_w, Lr_b,
           Lh_w, Lh_b, P1_w, P1_b, P2_w, P2_b, edge_index, node_ids):
    odim = P2_w.shape[1]
    src = edge_index[0].astype(jnp.int32)
    dst = edge_index[1].astype(jnp.int32)

    pad = EPAD - E
    pi = jnp.arange(pad, dtype=jnp.int32)
    src_p = jnp.concatenate([src, pi % N])
    dst_p = jnp.concatenate([dst, N + pi % (NP - N)])

    dst_a = dst_p.reshape(32, CA, 128)
    dst_c = dst_p.reshape(16, CC, 128)
    qoff = (jnp.arange(4, dtype=jnp.int32) // 2) * (2 * NP) \
        + jnp.arange(4, dtype=jnp.int32) % 2
    src_c = (2 * src_p.reshape(1, 16, CC, 128)
             + qoff[:, None, None, None]).reshape(64, CC, 128)

    ones_col = jnp.zeros((128, 16), _f32).at[:, 0].set(1.0)
    zeros16 = jnp.zeros((NP, 16), _f32)
    zeros64 = jnp.zeros((NP, HD), _f32)

    degp = _deg_kernel()(dst_a, ones_col, zeros16).reshape(2, NP, 16)

    x_pad = jnp.pad(node_embeddings, ((0, NP - N), (0, 0)))
    wcopy = lambda r: (0, 0)
    y2, base2 = pl.pallas_call(
        _b_body,
        grid=(GRID,),
        in_specs=[
            pl.BlockSpec((RB, D), lambda r: (r, 0)),
            pl.BlockSpec((D, D), wcopy),
            pl.BlockSpec((D, D), wcopy),
            pl.BlockSpec((1, D), wcopy),
            pl.BlockSpec((1, D), wcopy),
            pl.BlockSpec((2, RB, 16), lambda r: (0, r, 0)),
        ],
        out_specs=[
            pl.BlockSpec((2, RB, D), lambda r: (0, r, 0)),
            pl.BlockSpec((2, RB, D), lambda r: (0, r, 0)),
        ],
        out_shape=[
            jax.ShapeDtypeStruct((2, NP, D), _f32),
            jax.ShapeDtypeStruct((2, NP, D), _f32),
        ],
    )(x_pad, Wz, Wh, bz.reshape(1, D), bh.reshape(1, D), degp)

    acc = _conv_kernel()(y2.reshape(4 * NP, HD), src_c, dst_c, zeros64)
    acc = acc.reshape(2, NP, D)

    p2p = jnp.pad(P2_w, ((0, 0), (0, 16 - odim)))
    p2bp = jnp.pad(P2_b, (0, 16 - odim)).reshape(1, 16)
    logits_t = pl.pallas_call(
        _d_body,
        grid=(GRID,),
        in_specs=[
            pl.BlockSpec((2, RB, D), lambda r: (0, r, 0)),
            pl.BlockSpec((2, RB, D), lambda r: (0, r, 0)),
            pl.BlockSpec((2, RB, 16), lambda r: (0, r, 0)),
            pl.BlockSpec((D, D), wcopy),
            pl.BlockSpec((1, D), wcopy),
            pl.BlockSpec((D, D), wcopy),
            pl.BlockSpec((1, D), wcopy),
            pl.BlockSpec((D, D), wcopy),
            pl.BlockSpec((1, D), wcopy),
            pl.BlockSpec((D, 16), wcopy),
            pl.BlockSpec((1, 16), wcopy),
        ],
        out_specs=pl.BlockSpec((RB, 16), lambda r: (r, 0)),
        out_shape=jax.ShapeDtypeStruct((NP, 16), _f32),
    )(acc, base2, degp, Lz_w[:D], Lz_b.reshape(1, D), Lh_w[:D],
      Lh_b.reshape(1, D), P1_w, P1_b.reshape(1, D), p2p, p2bp)

    ids = node_ids.astype(jnp.int32).reshape(32, 128)
    out = _gather_kernel()(logits_t, ids)
    return out[:, :odim]


# speedup vs baseline: 37.8714x; 1.0683x over previous
import functools

import jax
import jax.numpy as jnp
from jax import lax
from jax.experimental import pallas as pl
from jax.experimental.pallas import tpu as pltpu
from jax.experimental.pallas import tpu_sc as plsc

N = 10000
D = 128
HD = D // 2
E = 320000
NB = 4096
NP = 10240
RPT = NP // 16
CA = 80
CC = 160
NBUF = 4
EPAD = 32 * CA * 128
RB = 1280
GRID = NP // RB

_f32 = jnp.float32
_SC_PARAMS = pltpu.CompilerParams(use_tc_tiling_on_sc=False)


def _sc_mesh():
    return plsc.VectorSubcoreMesh(
        core_axis_name="c", subcore_axis_name="s", num_cores=2, num_subcores=16
    )


def _deg_kernel():
    @functools.partial(
        pl.kernel,
        out_type=jax.ShapeDtypeStruct((2 * NP, 16), _f32),
        mesh=_sc_mesh(),
        scratch_types=[
            pltpu.VMEM((CA, 128), jnp.int32),
            pltpu.VMEM((128, 16), _f32),
            pltpu.VMEM_SHARED((NP, 16), _f32),
        ],
        compiler_params=_SC_PARAMS,
    )
    def deg_k(dst_hbm, ones_hbm, zeros_hbm, out_hbm, dst_v, ones_v, acc):
        c = lax.axis_index("c")
        s = lax.axis_index("s")
        wid = c * 16 + s
        pltpu.sync_copy(dst_hbm.at[wid], dst_v)
        pltpu.sync_copy(ones_hbm, ones_v)
        pltpu.sync_copy(zeros_hbm.at[pl.ds(s * RPT, RPT)],
                        acc.at[pl.ds(s * RPT, RPT)])
        plsc.subcore_barrier()

        def body(j, carry):
            pltpu.sync_copy(ones_v, acc.at[dst_v.at[j]], add=True)
            return carry

        lax.fori_loop(0, CA, body, 0)
        plsc.subcore_barrier()
        pltpu.sync_copy(acc.at[pl.ds(s * RPT, RPT)],
                        out_hbm.at[pl.ds(c * NP + s * RPT, RPT)])

    return deg_k


def _b_body(x_ref, wz_ref, wh_ref, bz_ref, bh_ref, degp_ref, y_ref, base_ref):
    deg = degp_ref[0, :, 0] + degp_ref[1, :, 0] + 1.0
    dinv = lax.rsqrt(deg)[:, None]
    x = x_ref[...]
    for k, (w_ref, b_ref) in enumerate(((wz_ref, bz_ref), (wh_ref, bh_ref))):
        xw = jnp.dot(x, w_ref[...], preferred_element_type=_f32)
        y_ref[k] = xw * dinv
        base_ref[k] = xw * (dinv * dinv) + b_ref[...]


def _conv_kernel():
    @functools.partial(
        pl.kernel,
        out_type=jax.ShapeDtypeStruct((2 * NP, D), _f32),
        mesh=_sc_mesh(),
        scratch_types=[
            pltpu.VMEM((CC, 128), jnp.int32),
            pltpu.VMEM((CC, 128), jnp.int32),
            [pltpu.VMEM((128, HD), _f32)] * NBUF,
            pltpu.VMEM_SHARED((NP, HD), _f32),
            [pltpu.SemaphoreType.DMA] * NBUF,
            [pltpu.SemaphoreType.DMA] * NBUF,
        ],
        compiler_params=_SC_PARAMS,
    )
    def conv_k(y_hbm, src_hbm, dst_hbm, zeros_hbm, out_hbm,
               src_v, dst_v, gb, acc, gs, ss):
        c = lax.axis_index("c")
        s = lax.axis_index("s")
        pltpu.sync_copy(dst_hbm.at[s], dst_v)
        for p in range(2):
            q = c * 2 + p
            pltpu.sync_copy(src_hbm.at[q * 16 + s], src_v)
            pltpu.sync_copy(zeros_hbm.at[pl.ds(s * RPT, RPT)],
                            acc.at[pl.ds(s * RPT, RPT)])
            plsc.subcore_barrier()

            for b in range(NBUF):
                pltpu.async_copy(y_hbm.at[src_v.at[b]], gb[b], gs[b])

            def blk(i, carry):
                base = NBUF * i
                for b in range(NBUF):
                    j = base + b
                    pltpu.make_async_copy(
                        y_hbm.at[src_v.at[j]], gb[b], gs[b]).wait()
                    pltpu.async_copy(
                        gb[b], acc.at[dst_v.at[j]], ss[b], add=True)
                for b in range(NBUF):
                    j = base + b
                    nxt = jnp.minimum(j + NBUF, CC - 1)
                    pltpu.make_async_copy(
                        gb[b], acc.at[dst_v.at[j]], ss[b]).wait()
                    pltpu.async_copy(y_hbm.at[src_v.at[nxt]], gb[b], gs[b])
                return carry

            lax.fori_loop(0, CC // NBUF, blk, 0)
            for b in range(NBUF):
                pltpu.make_async_copy(
                    y_hbm.at[src_v.at[CC - 1]], gb[b], gs[b]).wait()
            plsc.subcore_barrier()
            pltpu.sync_copy(
                acc.at[pl.ds(s * RPT, RPT)],
                out_hbm.at[pl.ds(c * NP + s * RPT, RPT), pl.ds(p * HD, HD)])
            plsc.subcore_barrier()

    return conv_k


def _d_body(acc_ref, base_ref, degp_ref, lz_ref, lzb_ref, lh_ref, lhb_ref,
            p1_ref, p1b_ref, p2_ref, p2b_ref, out_ref):
    deg = degp_ref[0, :, 0] + degp_ref[1, :, 0] + 1.0
    dinv = lax.rsqrt(deg)[:, None]
    cz = acc_ref[0] * dinv + base_ref[0]
    z = jax.nn.sigmoid(
        jnp.dot(cz, lz_ref[...], preferred_element_type=_f32) + lzb_ref[...])
    ch = acc_ref[1] * dinv + base_ref[1]
    ht = jnp.tanh(
        jnp.dot(ch, lh_ref[...], preferred_element_type=_f32) + lhb_ref[...])
    hn = (1.0 - z) * ht
    h = jax.nn.relu(
        jnp.dot(hn, p1_ref[...], preferred_element_type=_f32) + p1b_ref[...])
    out_ref[...] = (
        jnp.dot(h, p2_ref[...], preferred_element_type=_f32) + p2b_ref[...])


def _gather_kernel():
    @functools.partial(
        pl.kernel,
        out_type=jax.ShapeDtypeStruct((NB, 16), _f32),
        mesh=_sc_mesh(),
        scratch_types=[
            pltpu.VMEM((128,), jnp.int32),
            pltpu.VMEM((128, 16), _f32),
            pltpu.SemaphoreType.DMA,
        ],
        compiler_params=_SC_PARAMS,
    )
    def gather_k(tab_hbm, ids_hbm, out_hbm, idx_v, rows_v, sem):
        wid = lax.axis_index("c") * 16 + lax.axis_index("s")
        pltpu.sync_copy(ids_hbm.at[wid], idx_v)
        pltpu.async_copy(tab_hbm.at[idx_v], rows_v, sem).wait()
        pltpu.sync_copy(rows_v, out_hbm.at[pl.ds(wid * 128, 128)])

    return gather_k


def kernel(node_embeddings, Wz, bz, Wr, br, Wh, bh, Lz_w, Lz_b, Lr_w, Lr_b,
           Lh_w, Lh_b, P1_w, P1_b, P2_w, P2_b, edge_index, node_ids):
    odim = P2_w.shape[1]
    src = edge_index[0].astype(jnp.int32)
    dst = edge_index[1].astype(jnp.int32)

    pad = EPAD - E
    pi = jnp.arange(pad, dtype=jnp.int32)
    src_p = jnp.concatenate([src, pi % N])
    dst_p = jnp.concatenate([dst, N + pi % (NP - N)])

    dst_a = dst_p.reshape(32, CA, 128)
    dst_c = dst_p.reshape(16, CC, 128)
    qoff = (jnp.arange(4, dtype=jnp.int32) // 2) * (2 * NP) \
        + jnp.arange(4, dtype=jnp.int32) % 2
    src_c = (2 * src_p.reshape(1, 16, CC, 128)
             + qoff[:, None, None, None]).reshape(64, CC, 128)

    ones_col = jnp.zeros((128, 16), _f32).at[:, 0].set(1.0)
    zeros16 = jnp.zeros((NP, 16), _f32)
    zeros64 = jnp.zeros((NP, HD), _f32)

    degp = _deg_kernel()(dst_a, ones_col, zeros16).reshape(2, NP, 16)

    x_pad = jnp.pad(node_embeddings, ((0, NP - N), (0, 0)))
    wcopy = lambda r: (0, 0)
    y2, base2 = pl.pallas_call(
        _b_body,
        grid=(GRID,),
        in_specs=[
            pl.BlockSpec((RB, D), lambda r: (r, 0)),
            pl.BlockSpec((D, D), wcopy),
            pl.BlockSpec((D, D), wcopy),
            pl.BlockSpec((1, D), wcopy),
            pl.BlockSpec((1, D), wcopy),
            pl.BlockSpec((2, RB, 16), lambda r: (0, r, 0)),
        ],
        out_specs=[
            pl.BlockSpec((2, RB, D), lambda r: (0, r, 0)),
            pl.BlockSpec((2, RB, D), lambda r: (0, r, 0)),
        ],
        out_shape=[
            jax.ShapeDtypeStruct((2, NP, D), _f32),
            jax.ShapeDtypeStruct((2, NP, D), _f32),
        ],
    )(x_pad, Wz, Wh, bz.reshape(1, D), bh.reshape(1, D), degp)

    acc = _conv_kernel()(y2.reshape(4 * NP, HD), src_c, dst_c, zeros64)
    acc = acc.reshape(2, NP, D)

    p2p = jnp.pad(P2_w, ((0, 0), (0, 16 - odim)))
    p2bp = jnp.pad(P2_b, (0, 16 - odim)).reshape(1, 16)
    logits_t = pl.pallas_call(
        _d_body,
        grid=(GRID,),
        in_specs=[
            pl.BlockSpec((2, RB, D), lambda r: (0, r, 0)),
            pl.BlockSpec((2, RB, D), lambda r: (0, r, 0)),
            pl.BlockSpec((2, RB, 16), lambda r: (0, r, 0)),
            pl.BlockSpec((D, D), wcopy),
            pl.BlockSpec((1, D), wcopy),
            pl.BlockSpec((D, D), wcopy),
            pl.BlockSpec((1, D), wcopy),
            pl.BlockSpec((D, D), wcopy),
            pl.BlockSpec((1, D), wcopy),
            pl.BlockSpec((D, 16), wcopy),
            pl.BlockSpec((1, 16), wcopy),
        ],
        out_specs=pl.BlockSpec((RB, 16), lambda r: (r, 0)),
        out_shape=jax.ShapeDtypeStruct((NP, 16), _f32),
    )(acc, base2, degp, Lz_w[:D], Lz_b.reshape(1, D), Lh_w[:D],
      Lh_b.reshape(1, D), P1_w, P1_b.reshape(1, D), p2p, p2bp)

    ids = node_ids.astype(jnp.int32).reshape(32, 128)
    out = _gather_kernel()(logits_t, ids)
    return out[:, :odim]


# speedup vs baseline: 39.0224x; 1.0304x over previous
import functools

import jax
import jax.numpy as jnp
from jax import lax
from jax.experimental import pallas as pl
from jax.experimental.pallas import tpu as pltpu
from jax.experimental.pallas import tpu_sc as plsc

N = 10000
D = 128
HD = D // 2
E = 320000
NB = 4096
NP = 10240
RPT = NP // 16
CA = 80
CC = 160
NBUF = 5
EPAD = 32 * CA * 128
RB = 1280
GRID = NP // RB

_f32 = jnp.float32
_SC_PARAMS = pltpu.CompilerParams(use_tc_tiling_on_sc=False)


def _sc_mesh():
    return plsc.VectorSubcoreMesh(
        core_axis_name="c", subcore_axis_name="s", num_cores=2, num_subcores=16
    )


def _deg_kernel():
    @functools.partial(
        pl.kernel,
        out_type=jax.ShapeDtypeStruct((2 * NP, 16), _f32),
        mesh=_sc_mesh(),
        scratch_types=[
            pltpu.VMEM((CA, 128), jnp.int32),
            pltpu.VMEM((128, 16), _f32),
            pltpu.VMEM_SHARED((NP, 16), _f32),
        ],
        compiler_params=_SC_PARAMS,
    )
    def deg_k(dst_hbm, ones_hbm, zeros_hbm, out_hbm, dst_v, ones_v, acc):
        c = lax.axis_index("c")
        s = lax.axis_index("s")
        wid = c * 16 + s
        pltpu.sync_copy(dst_hbm.at[wid], dst_v)
        pltpu.sync_copy(ones_hbm, ones_v)
        pltpu.sync_copy(zeros_hbm.at[pl.ds(s * RPT, RPT)],
                        acc.at[pl.ds(s * RPT, RPT)])
        plsc.subcore_barrier()

        def body(j, carry):
            pltpu.sync_copy(ones_v, acc.at[dst_v.at[j]], add=True)
            return carry

        lax.fori_loop(0, CA, body, 0)
        plsc.subcore_barrier()
        pltpu.sync_copy(acc.at[pl.ds(s * RPT, RPT)],
                        out_hbm.at[pl.ds(c * NP + s * RPT, RPT)])

    return deg_k


def _b_body(x_ref, wz_ref, wh_ref, degp_ref, y_ref):
    deg = degp_ref[0, :, 0] + degp_ref[1, :, 0] + 1.0
    dinv = lax.rsqrt(deg)[:, None]
    x = x_ref[...]
    for k, w_ref in enumerate((wz_ref, wh_ref)):
        xw = jnp.dot(x, w_ref[...], preferred_element_type=_f32)
        y_ref[k] = xw * dinv


def _conv_kernel():
    @functools.partial(
        pl.kernel,
        out_type=jax.ShapeDtypeStruct((2 * NP, D), _f32),
        mesh=_sc_mesh(),
        scratch_types=[
            pltpu.VMEM((CC, 128), jnp.int32),
            pltpu.VMEM((CC, 128), jnp.int32),
            [pltpu.VMEM((128, HD), _f32)] * NBUF,
            pltpu.VMEM_SHARED((NP, HD), _f32),
            [pltpu.SemaphoreType.DMA] * NBUF,
            [pltpu.SemaphoreType.DMA] * NBUF,
        ],
        compiler_params=_SC_PARAMS,
    )
    def conv_k(y_hbm, src_hbm, dst_hbm, zeros_hbm, out_hbm,
               src_v, dst_v, gb, acc, gs, ss):
        c = lax.axis_index("c")
        s = lax.axis_index("s")
        pltpu.sync_copy(dst_hbm.at[s], dst_v)
        for p in range(2):
            q = c * 2 + p
            pltpu.sync_copy(src_hbm.at[q * 16 + s], src_v)
            pltpu.sync_copy(zeros_hbm.at[pl.ds(s * RPT, RPT)],
                            acc.at[pl.ds(s * RPT, RPT)])
            plsc.subcore_barrier()

            for b in range(NBUF):
                pltpu.async_copy(y_hbm.at[src_v.at[b]], gb[b], gs[b])

            def blk(i, carry):
                base = NBUF * i
                for b in range(NBUF):
                    j = base + b
                    pltpu.make_async_copy(
                        y_hbm.at[src_v.at[j]], gb[b], gs[b]).wait()
                    pltpu.async_copy(
                        gb[b], acc.at[dst_v.at[j]], ss[b], add=True)
                for b in range(NBUF):
                    j = base + b
                    nxt = jnp.minimum(j + NBUF, CC - 1)
                    pltpu.make_async_copy(
                        gb[b], acc.at[dst_v.at[j]], ss[b]).wait()
                    pltpu.async_copy(y_hbm.at[src_v.at[nxt]], gb[b], gs[b])
                return carry

            lax.fori_loop(0, CC // NBUF, blk, 0)
            for b in range(NBUF):
                pltpu.make_async_copy(
                    y_hbm.at[src_v.at[CC - 1]], gb[b], gs[b]).wait()
            plsc.subcore_barrier()
            pltpu.sync_copy(
                acc.at[pl.ds(s * RPT, RPT)],
                out_hbm.at[pl.ds(c * NP + s * RPT, RPT), pl.ds(p * HD, HD)])
            plsc.subcore_barrier()

    return conv_k


def _d_body(acc_ref, y_ref, degp_ref, bz_ref, bh_ref, lz_ref, lzb_ref,
            lh_ref, lhb_ref, p1_ref, p1b_ref, p2_ref, p2b_ref, out_ref):
    deg = degp_ref[0, :, 0] + degp_ref[1, :, 0] + 1.0
    dinv = lax.rsqrt(deg)[:, None]
    cz = (acc_ref[0] + y_ref[0]) * dinv + bz_ref[...]
    z = jax.nn.sigmoid(
        jnp.dot(cz, lz_ref[...], preferred_element_type=_f32) + lzb_ref[...])
    ch = (acc_ref[1] + y_ref[1]) * dinv + bh_ref[...]
    ht = jnp.tanh(
        jnp.dot(ch, lh_ref[...], preferred_element_type=_f32) + lhb_ref[...])
    hn = (1.0 - z) * ht
    h = jax.nn.relu(
        jnp.dot(hn, p1_ref[...], preferred_element_type=_f32) + p1b_ref[...])
    out_ref[...] = (
        jnp.dot(h, p2_ref[...], preferred_element_type=_f32) + p2b_ref[...])


def _gather_kernel():
    @functools.partial(
        pl.kernel,
        out_type=jax.ShapeDtypeStruct((NB, 16), _f32),
        mesh=_sc_mesh(),
        scratch_types=[
            pltpu.VMEM((128,), jnp.int32),
            pltpu.VMEM((128, 16), _f32),
            pltpu.SemaphoreType.DMA,
        ],
        compiler_params=_SC_PARAMS,
    )
    def gather_k(tab_hbm, ids_hbm, out_hbm, idx_v, rows_v, sem):
        wid = lax.axis_index("c") * 16 + lax.axis_index("s")
        pltpu.sync_copy(ids_hbm.at[wid], idx_v)
        pltpu.async_copy(tab_hbm.at[idx_v], rows_v, sem).wait()
        pltpu.sync_copy(rows_v, out_hbm.at[pl.ds(wid * 128, 128)])

    return gather_k


def kernel(node_embeddings, Wz, bz, Wr, br, Wh, bh, Lz_w, Lz_b, Lr_w, Lr_b,
           Lh_w, Lh_b, P1_w, P1_b, P2_w, P2_b, edge_index, node_ids):
    odim = P2_w.shape[1]
    src = edge_index[0].astype(jnp.int32)
    dst = edge_index[1].astype(jnp.int32)

    pad = EPAD - E
    pi = jnp.arange(pad, dtype=jnp.int32)
    src_p = jnp.concatenate([src, pi % N])
    dst_p = jnp.concatenate([dst, N + pi % (NP - N)])

    dst_a = dst_p.reshape(32, CA, 128)
    dst_c = dst_p.reshape(16, CC, 128)
    qoff = (jnp.arange(4, dtype=jnp.int32) // 2) * (2 * NP) \
        + jnp.arange(4, dtype=jnp.int32) % 2
    src_c = (2 * src_p.reshape(1, 16, CC, 128)
             + qoff[:, None, None, None]).reshape(64, CC, 128)

    ones_col = jnp.zeros((128, 16), _f32).at[:, 0].set(1.0)
    zeros16 = jnp.zeros((NP, 16), _f32)
    zeros64 = jnp.zeros((NP, HD), _f32)

    degp = _deg_kernel()(dst_a, ones_col, zeros16).reshape(2, NP, 16)

    wcopy = lambda r: (0, 0)
    y2 = pl.pallas_call(
        _b_body,
        grid=(GRID,),
        in_specs=[
            pl.BlockSpec((RB, D), lambda r: (r, 0)),
            pl.BlockSpec((D, D), wcopy),
            pl.BlockSpec((D, D), wcopy),
            pl.BlockSpec((2, RB, 16), lambda r: (0, r, 0)),
        ],
        out_specs=pl.BlockSpec((2, RB, D), lambda r: (0, r, 0)),
        out_shape=jax.ShapeDtypeStruct((2, NP, D), _f32),
    )(node_embeddings, Wz, Wh, degp)

    acc = _conv_kernel()(y2.reshape(4 * NP, HD), src_c, dst_c, zeros64)
    acc = acc.reshape(2, NP, D)

    p2p = jnp.pad(P2_w, ((0, 0), (0, 16 - odim)))
    p2bp = jnp.pad(P2_b, (0, 16 - odim)).reshape(1, 16)
    logits_t = pl.pallas_call(
        _d_body,
        grid=(GRID,),
        in_specs=[
            pl.BlockSpec((2, RB, D), lambda r: (0, r, 0)),
            pl.BlockSpec((2, RB, D), lambda r: (0, r, 0)),
            pl.BlockSpec((2, RB, 16), lambda r: (0, r, 0)),
            pl.BlockSpec((1, D), wcopy),
            pl.BlockSpec((1, D), wcopy),
            pl.BlockSpec((D, D), wcopy),
            pl.BlockSpec((1, D), wcopy),
            pl.BlockSpec((D, D), wcopy),
            pl.BlockSpec((1, D), wcopy),
            pl.BlockSpec((D, D), wcopy),
            pl.BlockSpec((1, D), wcopy),
            pl.BlockSpec((D, 16), wcopy),
            pl.BlockSpec((1, 16), wcopy),
        ],
        out_specs=pl.BlockSpec((RB, 16), lambda r: (r, 0)),
        out_shape=jax.ShapeDtypeStruct((NP, 16), _f32),
    )(acc, y2, degp, bz.reshape(1, D), bh.reshape(1, D), Lz_w[:D],
      Lz_b.reshape(1, D), Lh_w[:D], Lh_b.reshape(1, D), P1_w,
      P1_b.reshape(1, D), p2p, p2bp)

    ids = node_ids.astype(jnp.int32).reshape(32, 128)
    out = _gather_kernel()(logits_t, ids)
    return out[:, :odim]


# speedup vs baseline: 57.3294x; 1.4691x over previous
import functools

import jax
import jax.numpy as jnp
from jax import lax
from jax.experimental import pallas as pl
from jax.experimental.pallas import tpu as pltpu
from jax.experimental.pallas import tpu_sc as plsc

N = 10000
D = 128
HD = D // 2
E = 320000
NB = 4096
NP = 10240
RPT = NP // 16
CA = 80
CC = 80
NBUF = 5
EPAD = 32 * CA * 128
RB = 1280
GRID = NP // RB

_f32 = jnp.float32
_SC_PARAMS = pltpu.CompilerParams(use_tc_tiling_on_sc=False)


def _sc_mesh():
    return plsc.VectorSubcoreMesh(
        core_axis_name="c", subcore_axis_name="s", num_cores=2, num_subcores=16
    )


def _deg_kernel():
    @functools.partial(
        pl.kernel,
        out_type=jax.ShapeDtypeStruct((2 * NP, 16), _f32),
        mesh=_sc_mesh(),
        scratch_types=[
            pltpu.VMEM((CA, 128), jnp.int32),
            pltpu.VMEM((128, 16), _f32),
            pltpu.VMEM_SHARED((NP, 16), _f32),
            pltpu.SemaphoreType.DMA,
        ],
        compiler_params=_SC_PARAMS,
    )
    def deg_k(dst_hbm, ones_hbm, zeros_hbm, out_hbm, dst_v, ones_v, acc, sa):
        c = lax.axis_index("c")
        s = lax.axis_index("s")
        wid = c * 16 + s
        pltpu.sync_copy(dst_hbm.at[wid], dst_v)
        pltpu.sync_copy(ones_hbm, ones_v)
        pltpu.sync_copy(zeros_hbm.at[pl.ds(s * RPT, RPT)],
                        acc.at[pl.ds(s * RPT, RPT)])
        plsc.subcore_barrier()

        def body(j, carry):
            pltpu.async_copy(ones_v, acc.at[dst_v.at[j]], sa, add=True)
            return carry

        lax.fori_loop(0, CA, body, 0)

        def drain(j, carry):
            pltpu.make_async_copy(ones_v, acc.at[dst_v.at[0]], sa).wait()
            return carry

        lax.fori_loop(0, CA, drain, 0)
        plsc.subcore_barrier()
        pltpu.sync_copy(acc.at[pl.ds(s * RPT, RPT)],
                        out_hbm.at[pl.ds(c * NP + s * RPT, RPT)])

    return deg_k


def _b_body(x_ref, degp_ref, y_ref):
    deg = degp_ref[0, :, 0] + degp_ref[1, :, 0] + 1.0
    y_ref[...] = x_ref[...] * lax.rsqrt(deg)[:, None]


def _conv_kernel():
    @functools.partial(
        pl.kernel,
        out_type=jax.ShapeDtypeStruct((2 * NP, D), _f32),
        mesh=_sc_mesh(),
        scratch_types=[
            pltpu.VMEM((CC, 128), jnp.int32),
            pltpu.VMEM((CC, 128), jnp.int32),
            [pltpu.VMEM((128, HD), _f32)] * NBUF,
            pltpu.VMEM_SHARED((NP, HD), _f32),
            [pltpu.SemaphoreType.DMA] * NBUF,
            [pltpu.SemaphoreType.DMA] * NBUF,
        ],
        compiler_params=_SC_PARAMS,
    )
    def conv_k(y_hbm, src_hbm, dst_hbm, zeros_hbm, out_hbm,
               src_v, dst_v, gb, acc, gs, ss):
        c = lax.axis_index("c")
        s = lax.axis_index("s")
        wid = c * 16 + s
        pltpu.sync_copy(dst_hbm.at[wid], dst_v)
        for p in range(2):
            pltpu.sync_copy(src_hbm.at[p * 32 + wid], src_v)
            pltpu.sync_copy(zeros_hbm.at[pl.ds(s * RPT, RPT)],
                            acc.at[pl.ds(s * RPT, RPT)])
            plsc.subcore_barrier()

            for b in range(NBUF):
                pltpu.async_copy(y_hbm.at[src_v.at[b]], gb[b], gs[b])

            def blk(i, carry):
                base = NBUF * i
                for b in range(NBUF):
                    j = base + b
                    pltpu.make_async_copy(
                        y_hbm.at[src_v.at[j]], gb[b], gs[b]).wait()
                    pltpu.async_copy(
                        gb[b], acc.at[dst_v.at[j]], ss[b], add=True)
                for b in range(NBUF):
                    j = base + b
                    nxt = jnp.minimum(j + NBUF, CC - 1)
                    pltpu.make_async_copy(
                        gb[b], acc.at[dst_v.at[j]], ss[b]).wait()
                    pltpu.async_copy(y_hbm.at[src_v.at[nxt]], gb[b], gs[b])
                return carry

            lax.fori_loop(0, CC // NBUF, blk, 0)
            for b in range(NBUF):
                pltpu.make_async_copy(
                    y_hbm.at[src_v.at[CC - 1]], gb[b], gs[b]).wait()
            plsc.subcore_barrier()
            pltpu.sync_copy(
                acc.at[pl.ds(s * RPT, RPT)],
                out_hbm.at[pl.ds(c * NP + s * RPT, RPT), pl.ds(p * HD, HD)])
            plsc.subcore_barrier()

    return conv_k


def _d_body(acc_ref, x_ref, degp_ref, wz_ref, bz_ref, wh_ref, bh_ref,
            lz_ref, lzb_ref, lh_ref, lhb_ref, p1_ref, p1b_ref,
            p2_ref, p2b_ref, out_ref):
    deg = degp_ref[0, :, 0] + degp_ref[1, :, 0] + 1.0
    dinv = lax.rsqrt(deg)[:, None]
    t = (acc_ref[0] + acc_ref[1]) * dinv + x_ref[...] * (dinv * dinv)
    cz = jnp.dot(t, wz_ref[...], preferred_element_type=_f32) + bz_ref[...]
    z = jax.nn.sigmoid(
        jnp.dot(cz, lz_ref[...], preferred_element_type=_f32) + lzb_ref[...])
    ch = jnp.dot(t, wh_ref[...], preferred_element_type=_f32) + bh_ref[...]
    ht = jnp.tanh(
        jnp.dot(ch, lh_ref[...], preferred_element_type=_f32) + lhb_ref[...])
    hn = (1.0 - z) * ht
    h = jax.nn.relu(
        jnp.dot(hn, p1_ref[...], preferred_element_type=_f32) + p1b_ref[...])
    out_ref[...] = (
        jnp.dot(h, p2_ref[...], preferred_element_type=_f32) + p2b_ref[...])


def _gather_kernel():
    @functools.partial(
        pl.kernel,
        out_type=jax.ShapeDtypeStruct((NB, 16), _f32),
        mesh=_sc_mesh(),
        scratch_types=[
            pltpu.VMEM((128,), jnp.int32),
            pltpu.VMEM((128, 16), _f32),
            pltpu.SemaphoreType.DMA,
        ],
        compiler_params=_SC_PARAMS,
    )
    def gather_k(tab_hbm, ids_hbm, out_hbm, idx_v, rows_v, sem):
        wid = lax.axis_index("c") * 16 + lax.axis_index("s")
        pltpu.sync_copy(ids_hbm.at[wid], idx_v)
        pltpu.async_copy(tab_hbm.at[idx_v], rows_v, sem).wait()
        pltpu.sync_copy(rows_v, out_hbm.at[pl.ds(wid * 128, 128)])

    return gather_k


def kernel(node_embeddings, Wz, bz, Wr, br, Wh, bh, Lz_w, Lz_b, Lr_w, Lr_b,
           Lh_w, Lh_b, P1_w, P1_b, P2_w, P2_b, edge_index, node_ids):
    odim = P2_w.shape[1]
    src = edge_index[0].astype(jnp.int32)
    dst = edge_index[1].astype(jnp.int32)

    pad = EPAD - E
    pi = jnp.arange(pad, dtype=jnp.int32)
    src_p = jnp.concatenate([src, pi % N])
    dst_p = jnp.concatenate([dst, N + pi % (NP - N)])

    dst_a = dst_p.reshape(32, CA, 128)
    src_c = (2 * src_p.reshape(1, 32, CC, 128)
             + jnp.arange(2, dtype=jnp.int32)[:, None, None, None]
             ).reshape(64, CC, 128)

    ones_col = jnp.zeros((128, 16), _f32).at[:, 0].set(1.0)
    zeros16 = jnp.zeros((NP, 16), _f32)
    zeros_hd = jnp.zeros((NP, HD), _f32)

    degp = _deg_kernel()(dst_a, ones_col, zeros16).reshape(2, NP, 16)

    wcopy = lambda r: (0, 0)
    y = pl.pallas_call(
        _b_body,
        grid=(GRID,),
        in_specs=[
            pl.BlockSpec((RB, D), lambda r: (r, 0)),
            pl.BlockSpec((2, RB, 16), lambda r: (0, r, 0)),
        ],
        out_specs=pl.BlockSpec((RB, D), lambda r: (r, 0)),
        out_shape=jax.ShapeDtypeStruct((NP, D), _f32),
    )(node_embeddings, degp)

    acc = _conv_kernel()(y.reshape(2 * NP, HD), src_c, dst_a, zeros_hd)
    acc = acc.reshape(2, NP, D)

    p2p = jnp.pad(P2_w, ((0, 0), (0, 16 - odim)))
    p2bp = jnp.pad(P2_b, (0, 16 - odim)).reshape(1, 16)
    logits_t = pl.pallas_call(
        _d_body,
        grid=(GRID,),
        in_specs=[
            pl.BlockSpec((2, RB, D), lambda r: (0, r, 0)),
            pl.BlockSpec((RB, D), lambda r: (r, 0)),
            pl.BlockSpec((2, RB, 16), lambda r: (0, r, 0)),
            pl.BlockSpec((D, D), wcopy),
            pl.BlockSpec((1, D), wcopy),
            pl.BlockSpec((D, D), wcopy),
            pl.BlockSpec((1, D), wcopy),
            pl.BlockSpec((D, D), wcopy),
            pl.BlockSpec((1, D), wcopy),
            pl.BlockSpec((D, D), wcopy),
            pl.BlockSpec((1, D), wcopy),
            pl.BlockSpec((D, D), wcopy),
            pl.BlockSpec((1, D), wcopy),
            pl.BlockSpec((D, 16), wcopy),
            pl.BlockSpec((1, 16), wcopy),
        ],
        out_specs=pl.BlockSpec((RB, 16), lambda r: (r, 0)),
        out_shape=jax.ShapeDtypeStruct((NP, 16), _f32),
    )(acc, node_embeddings, degp, Wz, bz.reshape(1, D), Wh,
      bh.reshape(1, D), Lz_w[:D], Lz_b.reshape(1, D), Lh_w[:D],
      Lh_b.reshape(1, D), P1_w, P1_b.reshape(1, D), p2p, p2bp)

    ids = node_ids.astype(jnp.int32).reshape(32, 128)
    out = _gather_kernel()(logits_t, ids)
    return out[:, :odim]
